# baseline (speedup 1.0000x reference)
import functools

import jax
import jax.numpy as jnp
from jax import lax
from jax.experimental import pallas as pl
from jax.experimental.pallas import tpu as pltpu
from jax.experimental.pallas import tpu_sc as plsc

NUM_E = 64
K2 = 2
DIM = 768
NTOK = 4096
NASSIGN = NTOK * K2
BLK = 128
NBLK = NASSIGN // BLK + NUM_E
NPAD = NBLK * BLK


def _router_body(x_ref, q_ref, wg_ref, tg_ref, idx_ref, g_ref, loss_ref):
    logits = (
        jnp.dot(x_ref[...], wg_ref[...], preferred_element_type=jnp.float32)
        + jnp.dot(q_ref[...], tg_ref[...], preferred_element_type=jnp.float32)
    )
    cols = lax.broadcasted_iota(jnp.int32, logits.shape, 1)
    a1 = jnp.argmax(logits, axis=1).astype(jnp.int32)
    m1 = jnp.max(logits, axis=1)
    masked = jnp.where(cols == a1[:, None], -jnp.inf, logits)
    a2 = jnp.argmax(masked, axis=1).astype(jnp.int32)
    m2 = jnp.max(masked, axis=1)
    e2 = jnp.exp(m2 - m1)
    denom = 1.0 + e2
    g1 = 1.0 / denom
    g2 = e2 / denom

    idx_ref[...] = jnp.concatenate([a1[:, None], a2[:, None]], axis=1)
    g_ref[...] = jnp.concatenate([g1[:, None], g2[:, None]], axis=1)

    gates = jnp.where(cols == a1[:, None], g1[:, None], 0.0) + jnp.where(
        cols == a2[:, None], g2[:, None], 0.0
    )
    importance = jnp.sum(gates, axis=0)
    load = jnp.sum((gates > 0.0).astype(jnp.float32), axis=0)

    def cv2(v):
        n = float(NUM_E)
        mean = jnp.sum(v) / n
        var = jnp.sum((v - mean) ** 2) / (n - 1.0)
        return var / (mean * mean + 1e-10)

    loss = (cv2(importance) + cv2(load)) * 0.01
    loss_ref[...] = jnp.full((1, 1), loss, dtype=jnp.float32)


def _router(x, query, w_gate, task_gate, *, interpret=False):
    return pl.pallas_call(
        _router_body,
        out_shape=(
            jax.ShapeDtypeStruct((NTOK, K2), jnp.int32),
            jax.ShapeDtypeStruct((NTOK, K2), jnp.float32),
            jax.ShapeDtypeStruct((1, 1), jnp.float32),
        ),
        interpret=interpret,
    )(x, query, w_gate, task_gate)


def _ffn_body(ea_ref, xs_ref, w1_ref, b1_ref, w2_ref, b2_ref, g_ref, out_ref):
    xb = xs_ref[...]
    h = lax.dot_general(
        xb, w1_ref[0], (((1,), (1,)), ((), ())), preferred_element_type=jnp.float32
    )
    h = jnp.maximum(h + b1_ref[0], 0.0)
    o = lax.dot_general(
        h, w2_ref[0], (((1,), (1,)), ((), ())), preferred_element_type=jnp.float32
    )
    out_ref[...] = (o + b2_ref[0]) * g_ref[...]


def _ffn(blk_expert, xs, W1, b1, W2, b2, g_pad, *, interpret=False):
    grid_spec = pltpu.PrefetchScalarGridSpec(
        num_scalar_prefetch=1,
        grid=(NBLK,),
        in_specs=[
            pl.BlockSpec((BLK, DIM), lambda i, ea: (i, 0)),
            pl.BlockSpec((1, DIM, DIM), lambda i, ea: (ea[i], 0, 0)),
            pl.BlockSpec((1, 1, DIM), lambda i, ea: (ea[i], 0, 0)),
            pl.BlockSpec((1, DIM, DIM), lambda i, ea: (ea[i], 0, 0)),
            pl.BlockSpec((1, 1, DIM), lambda i, ea: (ea[i], 0, 0)),
            pl.BlockSpec((BLK, 1), lambda i, ea: (i, 0)),
        ],
        out_specs=pl.BlockSpec((BLK, DIM), lambda i, ea: (i, 0)),
    )
    return pl.pallas_call(
        _ffn_body,
        grid_spec=grid_spec,
        out_shape=jax.ShapeDtypeStruct((NPAD, DIM), jnp.float32),
        interpret=interpret,
    )(
        blk_expert,
        xs,
        W1,
        b1.reshape(NUM_E, 1, DIM),
        W2,
        b2.reshape(NUM_E, 1, DIM),
        g_pad,
    )


_SC_CHUNK = 64


def _gather_rows(table, idx):
    m = idx.shape[0]
    nc, ns = 2, 16
    nw = nc * ns
    per_w = m // nw
    assert per_w % _SC_CHUNK == 0
    mesh = plsc.VectorSubcoreMesh(core_axis_name="c", subcore_axis_name="s")

    def body(table_hbm, idx_hbm, out_hbm, idx_v, rows_v, sem):
        wid = lax.axis_index("s") * nc + lax.axis_index("c")
        base = wid * per_w

        def chunk(i, carry):
            b = base + i * _SC_CHUNK
            pltpu.sync_copy(idx_hbm.at[pl.ds(b, _SC_CHUNK)], idx_v)
            pltpu.async_copy(table_hbm.at[idx_v], rows_v, sem).wait()
            pltpu.sync_copy(rows_v, out_hbm.at[pl.ds(b, _SC_CHUNK)])
            return carry

        lax.fori_loop(0, per_w // _SC_CHUNK, chunk, 0)

    return pl.kernel(
        body,
        out_type=jax.ShapeDtypeStruct((m, DIM), jnp.float32),
        mesh=mesh,
        scratch_types=[
            pltpu.VMEM((_SC_CHUNK,), jnp.int32),
            pltpu.VMEM((_SC_CHUNK, DIM), jnp.float32),
            pltpu.SemaphoreType.DMA,
        ],
    )(table, idx)


def _route_meta(idx, g):
    i32 = jnp.int32
    flat_e = idx.reshape(-1)
    order = jnp.argsort(flat_e, stable=True).astype(i32)
    se = jnp.take(flat_e, order)
    st = (order // K2).astype(i32)
    gs = jnp.take(g.reshape(-1), order)
    off = jnp.searchsorted(se, jnp.arange(NUM_E + 1, dtype=i32)).astype(i32)
    cnt = off[1:] - off[:-1]
    nb = (cnt + BLK - 1) // BLK
    blk_start = jnp.concatenate(
        [jnp.zeros((1,), i32), jnp.cumsum(nb).astype(i32)]
    )[:NUM_E]
    blk_expert = jnp.repeat(
        jnp.arange(NUM_E, dtype=i32), nb, total_repeat_length=NBLK
    )
    p = jnp.arange(NPAD, dtype=i32)
    e_of_p = blk_expert[p // BLK]
    r = p - blk_start[e_of_p] * BLK
    j = off[e_of_p] + r
    valid = r < cnt[e_of_p]
    jc = jnp.clip(j, 0, NASSIGN - 1)
    st_pad = jnp.where(valid, st[jc], 0)
    g_pad = jnp.where(valid, gs[jc], 0.0)
    p_sorted = blk_start[se] * BLK + (jnp.arange(NASSIGN, dtype=i32) - off[se])
    inv = jnp.argsort(order).astype(i32)
    pos_flat = jnp.take(p_sorted, inv)
    posA = pos_flat[0::2]
    posB = pos_flat[1::2]
    return blk_expert, st_pad, g_pad, posA, posB


@jax.jit
def kernel(query, x, w_gate, task_gate, W1, b1, W2, b2):
    idx, g, loss = _router(x, query, w_gate, task_gate)
    blk_expert, st_pad, g_pad, posA, posB = _route_meta(idx, g)
    xs = _gather_rows(x, st_pad)
    os = _ffn(blk_expert, xs, W1, b1, W2, b2, g_pad.reshape(NPAD, 1))
    ab = _gather_rows(os, jnp.concatenate([posA, posB]))
    y = ab[:NTOK] + ab[NTOK:]
    return (y, loss[0, 0])


# speedup vs baseline: 1.0027x; 1.0027x over previous
import functools

import jax
import jax.numpy as jnp
from jax import lax
from jax.experimental import pallas as pl
from jax.experimental.pallas import tpu as pltpu
from jax.experimental.pallas import tpu_sc as plsc

NUM_E = 64
K2 = 2
DIM = 768
NTOK = 4096
NASSIGN = NTOK * K2
BLK = 128
NBLK = NASSIGN // BLK + NUM_E
NPAD = NBLK * BLK


def _router_body(x_ref, q_ref, wg_ref, tg_ref, idx_ref, g_ref, loss_ref):
    logits = (
        jnp.dot(x_ref[...], wg_ref[...], preferred_element_type=jnp.float32)
        + jnp.dot(q_ref[...], tg_ref[...], preferred_element_type=jnp.float32)
    )
    cols = lax.broadcasted_iota(jnp.int32, logits.shape, 1)
    a1 = jnp.argmax(logits, axis=1).astype(jnp.int32)
    m1 = jnp.max(logits, axis=1)
    masked = jnp.where(cols == a1[:, None], -jnp.inf, logits)
    a2 = jnp.argmax(masked, axis=1).astype(jnp.int32)
    m2 = jnp.max(masked, axis=1)
    e2 = jnp.exp(m2 - m1)
    denom = 1.0 + e2
    g1 = 1.0 / denom
    g2 = e2 / denom

    idx_ref[...] = jnp.concatenate([a1[:, None], a2[:, None]], axis=1)
    g_ref[...] = jnp.concatenate([g1[:, None], g2[:, None]], axis=1)

    gates = jnp.where(cols == a1[:, None], g1[:, None], 0.0) + jnp.where(
        cols == a2[:, None], g2[:, None], 0.0
    )
    importance = jnp.sum(gates, axis=0)
    load = jnp.sum((gates > 0.0).astype(jnp.float32), axis=0)

    def cv2(v):
        n = float(NUM_E)
        mean = jnp.sum(v) / n
        var = jnp.sum((v - mean) ** 2) / (n - 1.0)
        return var / (mean * mean + 1e-10)

    loss = (cv2(importance) + cv2(load)) * 0.01
    loss_ref[...] = jnp.full((1, 1), loss, dtype=jnp.float32)


def _router(x, query, w_gate, task_gate, *, interpret=False):
    return pl.pallas_call(
        _router_body,
        out_shape=(
            jax.ShapeDtypeStruct((NTOK, K2), jnp.int32),
            jax.ShapeDtypeStruct((NTOK, K2), jnp.float32),
            jax.ShapeDtypeStruct((1, 1), jnp.float32),
        ),
        interpret=interpret,
    )(x, query, w_gate, task_gate)


def _ffn_body(ea_ref, xs_ref, w1_ref, b1_ref, w2_ref, b2_ref, g_ref, out_ref):
    xb = xs_ref[...]
    h = lax.dot_general(
        xb, w1_ref[0], (((1,), (1,)), ((), ())), preferred_element_type=jnp.float32
    )
    h = jnp.maximum(h + b1_ref[0], 0.0)
    o = lax.dot_general(
        h, w2_ref[0], (((1,), (1,)), ((), ())), preferred_element_type=jnp.float32
    )
    out_ref[...] = (o + b2_ref[0]) * g_ref[...]


def _ffn(blk_expert, xs, W1, b1, W2, b2, g_pad, *, interpret=False):
    grid_spec = pltpu.PrefetchScalarGridSpec(
        num_scalar_prefetch=1,
        grid=(NBLK,),
        in_specs=[
            pl.BlockSpec((BLK, DIM), lambda i, ea: (i, 0)),
            pl.BlockSpec((1, DIM, DIM), lambda i, ea: (ea[i], 0, 0)),
            pl.BlockSpec((1, 1, DIM), lambda i, ea: (ea[i], 0, 0)),
            pl.BlockSpec((1, DIM, DIM), lambda i, ea: (ea[i], 0, 0)),
            pl.BlockSpec((1, 1, DIM), lambda i, ea: (ea[i], 0, 0)),
            pl.BlockSpec((BLK, 1), lambda i, ea: (i, 0)),
        ],
        out_specs=pl.BlockSpec((BLK, DIM), lambda i, ea: (i, 0)),
    )
    return pl.pallas_call(
        _ffn_body,
        grid_spec=grid_spec,
        out_shape=jax.ShapeDtypeStruct((NPAD, DIM), jnp.float32),
        interpret=interpret,
    )(
        blk_expert,
        xs,
        W1,
        b1.reshape(NUM_E, 1, DIM),
        W2,
        b2.reshape(NUM_E, 1, DIM),
        g_pad,
    )


_SC_CHUNK = 64


def _gather_rows(table, idx):
    m = idx.shape[0]
    nc, ns = 2, 16
    nw = nc * ns
    per_w = m // nw
    assert per_w % _SC_CHUNK == 0
    mesh = plsc.VectorSubcoreMesh(core_axis_name="c", subcore_axis_name="s")

    nch = per_w // _SC_CHUNK

    def body(table_hbm, idx_hbm, out_hbm, idx_v, rows0, rows1, sem0, sem1):
        wid = lax.axis_index("s") * nc + lax.axis_index("c")
        base = wid * per_w
        pltpu.sync_copy(idx_hbm.at[pl.ds(base, per_w)], idx_v)
        rows = (rows0, rows1)
        sems = (sem0, sem1)
        cps = {}
        for i in range(nch):
            b = i % 2
            cps[i] = pltpu.async_copy(
                table_hbm.at[idx_v.at[pl.ds(i * _SC_CHUNK, _SC_CHUNK)]],
                rows[b],
                sems[b],
            )
            if i >= 1:
                cps[i - 1].wait()
                pltpu.sync_copy(
                    rows[(i - 1) % 2],
                    out_hbm.at[pl.ds(base + (i - 1) * _SC_CHUNK, _SC_CHUNK)],
                )
        cps[nch - 1].wait()
        pltpu.sync_copy(
            rows[(nch - 1) % 2],
            out_hbm.at[pl.ds(base + (nch - 1) * _SC_CHUNK, _SC_CHUNK)],
        )

    return pl.kernel(
        body,
        out_type=jax.ShapeDtypeStruct((m, DIM), jnp.float32),
        mesh=mesh,
        scratch_types=[
            pltpu.VMEM((per_w,), jnp.int32),
            pltpu.VMEM((_SC_CHUNK, DIM), jnp.float32),
            pltpu.VMEM((_SC_CHUNK, DIM), jnp.float32),
            pltpu.SemaphoreType.DMA,
            pltpu.SemaphoreType.DMA,
        ],
    )(table, idx)


def _route_meta(idx, g):
    i32 = jnp.int32
    flat_e = idx.reshape(-1)
    order = jnp.argsort(flat_e, stable=True).astype(i32)
    se = jnp.take(flat_e, order)
    st = (order // K2).astype(i32)
    gs = jnp.take(g.reshape(-1), order)
    off = jnp.searchsorted(se, jnp.arange(NUM_E + 1, dtype=i32)).astype(i32)
    cnt = off[1:] - off[:-1]
    nb = (cnt + BLK - 1) // BLK
    blk_start = jnp.concatenate(
        [jnp.zeros((1,), i32), jnp.cumsum(nb).astype(i32)]
    )[:NUM_E]
    blk_expert = jnp.repeat(
        jnp.arange(NUM_E, dtype=i32), nb, total_repeat_length=NBLK
    )
    p = jnp.arange(NPAD, dtype=i32)
    e_of_p = blk_expert[p // BLK]
    r = p - blk_start[e_of_p] * BLK
    j = off[e_of_p] + r
    valid = r < cnt[e_of_p]
    jc = jnp.clip(j, 0, NASSIGN - 1)
    st_pad = jnp.where(valid, st[jc], 0)
    g_pad = jnp.where(valid, gs[jc], 0.0)
    p_sorted = blk_start[se] * BLK + (jnp.arange(NASSIGN, dtype=i32) - off[se])
    inv = jnp.argsort(order).astype(i32)
    pos_flat = jnp.take(p_sorted, inv)
    posA = pos_flat[0::2]
    posB = pos_flat[1::2]
    return blk_expert, st_pad, g_pad, posA, posB


@jax.jit
def kernel(query, x, w_gate, task_gate, W1, b1, W2, b2):
    idx, g, loss = _router(x, query, w_gate, task_gate)
    blk_expert, st_pad, g_pad, posA, posB = _route_meta(idx, g)
    xs = _gather_rows(x, st_pad)
    os = _ffn(blk_expert, xs, W1, b1, W2, b2, g_pad.reshape(NPAD, 1))
    ab = _gather_rows(os, jnp.concatenate([posA, posB]))
    y = ab[:NTOK] + ab[NTOK:]
    return (y, loss[0, 0])


# speedup vs baseline: 1.2235x; 1.2201x over previous
import functools

import jax
import jax.numpy as jnp
from jax import lax
from jax.experimental import pallas as pl
from jax.experimental.pallas import tpu as pltpu
from jax.experimental.pallas import tpu_sc as plsc

NUM_E = 64
K2 = 2
DIM = 768
NTOK = 4096
NASSIGN = NTOK * K2
BLK = 128
NBLK = NASSIGN // BLK + NUM_E
NPAD = NBLK * BLK


def _router_body(x_ref, q_ref, wg_ref, tg_ref, idx_ref, g_ref, loss_ref):
    logits = (
        jnp.dot(x_ref[...], wg_ref[...], preferred_element_type=jnp.float32)
        + jnp.dot(q_ref[...], tg_ref[...], preferred_element_type=jnp.float32)
    )
    cols = lax.broadcasted_iota(jnp.int32, logits.shape, 1)
    a1 = jnp.argmax(logits, axis=1).astype(jnp.int32)
    m1 = jnp.max(logits, axis=1)
    masked = jnp.where(cols == a1[:, None], -jnp.inf, logits)
    a2 = jnp.argmax(masked, axis=1).astype(jnp.int32)
    m2 = jnp.max(masked, axis=1)
    e2 = jnp.exp(m2 - m1)
    denom = 1.0 + e2
    g1 = 1.0 / denom
    g2 = e2 / denom

    idx_ref[...] = jnp.concatenate([a1[:, None], a2[:, None]], axis=1)
    g_ref[...] = jnp.concatenate([g1[:, None], g2[:, None]], axis=1)

    gates = jnp.where(cols == a1[:, None], g1[:, None], 0.0) + jnp.where(
        cols == a2[:, None], g2[:, None], 0.0
    )
    importance = jnp.sum(gates, axis=0)
    load = jnp.sum((gates > 0.0).astype(jnp.float32), axis=0)

    def cv2(v):
        n = float(NUM_E)
        mean = jnp.sum(v) / n
        var = jnp.sum((v - mean) ** 2) / (n - 1.0)
        return var / (mean * mean + 1e-10)

    loss = (cv2(importance) + cv2(load)) * 0.01
    loss_ref[...] = jnp.full((1, 1), loss, dtype=jnp.float32)


def _router(x, query, w_gate, task_gate, *, interpret=False):
    return pl.pallas_call(
        _router_body,
        out_shape=(
            jax.ShapeDtypeStruct((NTOK, K2), jnp.int32),
            jax.ShapeDtypeStruct((NTOK, K2), jnp.float32),
            jax.ShapeDtypeStruct((1, 1), jnp.float32),
        ),
        interpret=interpret,
    )(x, query, w_gate, task_gate)


def _ffn_body(ea_ref, xs_ref, w1_ref, b1_ref, w2_ref, b2_ref, g_ref, out_ref):
    xb = xs_ref[...]
    h = lax.dot_general(
        xb, w1_ref[0], (((1,), (1,)), ((), ())), preferred_element_type=jnp.float32
    )
    h = jnp.maximum(h + b1_ref[0], 0.0)
    o = lax.dot_general(
        h, w2_ref[0], (((1,), (1,)), ((), ())), preferred_element_type=jnp.float32
    )
    out_ref[...] = (o + b2_ref[0]) * g_ref[...]


def _ffn(blk_expert, xs, W1, b1, W2, b2, g_pad, *, interpret=False):
    grid_spec = pltpu.PrefetchScalarGridSpec(
        num_scalar_prefetch=1,
        grid=(NBLK,),
        in_specs=[
            pl.BlockSpec((BLK, DIM), lambda i, ea: (i, 0)),
            pl.BlockSpec((1, DIM, DIM), lambda i, ea: (ea[i], 0, 0)),
            pl.BlockSpec((1, 1, DIM), lambda i, ea: (ea[i], 0, 0)),
            pl.BlockSpec((1, DIM, DIM), lambda i, ea: (ea[i], 0, 0)),
            pl.BlockSpec((1, 1, DIM), lambda i, ea: (ea[i], 0, 0)),
            pl.BlockSpec((BLK, 1), lambda i, ea: (i, 0)),
        ],
        out_specs=pl.BlockSpec((BLK, DIM), lambda i, ea: (i, 0)),
    )
    return pl.pallas_call(
        _ffn_body,
        grid_spec=grid_spec,
        out_shape=jax.ShapeDtypeStruct((NPAD, DIM), jnp.float32),
        interpret=interpret,
    )(
        blk_expert,
        xs,
        W1,
        b1.reshape(NUM_E, 1, DIM),
        W2,
        b2.reshape(NUM_E, 1, DIM),
        g_pad,
    )


_SC_CHUNK = 64


def _gather_rows(table, idx):
    m = idx.shape[0]
    nc, ns = 2, 16
    nw = nc * ns
    per_w = m // nw
    assert per_w % _SC_CHUNK == 0
    mesh = plsc.VectorSubcoreMesh(core_axis_name="c", subcore_axis_name="s")

    nch = per_w // _SC_CHUNK

    def body(table_hbm, idx_hbm, out_hbm, idx_v, rows0, rows1, sem0, sem1):
        wid = lax.axis_index("s") * nc + lax.axis_index("c")
        base = wid * per_w
        pltpu.sync_copy(idx_hbm.at[pl.ds(base, per_w)], idx_v)
        rows = (rows0, rows1)
        sems = (sem0, sem1)
        cps = {}
        for i in range(nch):
            b = i % 2
            cps[i] = pltpu.async_copy(
                table_hbm.at[idx_v.at[pl.ds(i * _SC_CHUNK, _SC_CHUNK)]],
                rows[b],
                sems[b],
            )
            if i >= 1:
                cps[i - 1].wait()
                pltpu.sync_copy(
                    rows[(i - 1) % 2],
                    out_hbm.at[pl.ds(base + (i - 1) * _SC_CHUNK, _SC_CHUNK)],
                )
        cps[nch - 1].wait()
        pltpu.sync_copy(
            rows[(nch - 1) % 2],
            out_hbm.at[pl.ds(base + (nch - 1) * _SC_CHUNK, _SC_CHUNK)],
        )

    return pl.kernel(
        body,
        out_type=jax.ShapeDtypeStruct((m, DIM), jnp.float32),
        mesh=mesh,
        scratch_types=[
            pltpu.VMEM((per_w,), jnp.int32),
            pltpu.VMEM((_SC_CHUNK, DIM), jnp.float32),
            pltpu.VMEM((_SC_CHUNK, DIM), jnp.float32),
            pltpu.SemaphoreType.DMA,
            pltpu.SemaphoreType.DMA,
        ],
    )(table, idx)


def _route_meta(idx, g):
    i32 = jnp.int32
    flat_e = idx.reshape(-1)
    order = jnp.argsort(flat_e, stable=True).astype(i32)
    se = jnp.take(flat_e, order)
    st = (order // K2).astype(i32)
    gs = jnp.take(g.reshape(-1), order)
    off = jnp.searchsorted(se, jnp.arange(NUM_E + 1, dtype=i32)).astype(i32)
    cnt = off[1:] - off[:-1]
    nb = (cnt + BLK - 1) // BLK
    blk_start = jnp.concatenate(
        [jnp.zeros((1,), i32), jnp.cumsum(nb).astype(i32)]
    )[:NUM_E]
    blk_expert = jnp.repeat(
        jnp.arange(NUM_E, dtype=i32), nb, total_repeat_length=NBLK
    )
    p = jnp.arange(NPAD, dtype=i32)
    e_of_p = blk_expert[p // BLK]
    r = p - blk_start[e_of_p] * BLK
    j = off[e_of_p] + r
    valid = r < cnt[e_of_p]
    jc = jnp.clip(j, 0, NASSIGN - 1)
    st_pad = jnp.where(valid, st[jc], p & (NTOK - 1))
    g_pad = jnp.where(valid, gs[jc], 0.0)
    p_sorted = blk_start[se] * BLK + (jnp.arange(NASSIGN, dtype=i32) - off[se])
    inv = jnp.argsort(order).astype(i32)
    pos_flat = jnp.take(p_sorted, inv)
    posA = pos_flat[0::2]
    posB = pos_flat[1::2]
    return blk_expert, st_pad, g_pad, posA, posB


@jax.jit
def kernel(query, x, w_gate, task_gate, W1, b1, W2, b2):
    idx, g, loss = _router(x, query, w_gate, task_gate)
    blk_expert, st_pad, g_pad, posA, posB = _route_meta(idx, g)
    xs = _gather_rows(x, st_pad)
    os = _ffn(blk_expert, xs, W1, b1, W2, b2, g_pad.reshape(NPAD, 1))
    ab = _gather_rows(os, jnp.concatenate([posA, posB]))
    y = ab[:NTOK] + ab[NTOK:]
    return (y, loss[0, 0])
